# baseline (speedup 1.0000x reference)
import functools

import jax
import jax.numpy as jnp
from jax import lax
from jax.experimental import pallas as pl
from jax.experimental.pallas import tpu as pltpu
from jax.experimental.pallas import tpu_sc as plsc

N_ATOMS = 10000
N_EDGES = 320000
HIDDEN = 128
STEPS = 3

NC, NS = 2, 16
NW = NC * NS
EPW = N_EDGES // NW
CH = 40
NCHUNK = EPW // CH
NPAIR = NCHUNK // 2
assert NCHUNK == 2 * NPAIR
APT = 624
ATL = N_ATOMS - NS * APT
HL = HIDDEN // 16

_mesh = plsc.VectorSubcoreMesh(core_axis_name="c", subcore_axis_name="s")


def _zero_acc(zeros_hbm, acc, sid):
    pltpu.sync_copy(zeros_hbm.at[pl.ds(sid * APT, APT)],
                    acc.at[pl.ds(sid * APT, APT)])

    @pl.when(sid == NS - 1)
    def _():
        pltpu.sync_copy(zeros_hbm.at[pl.ds(NS * APT, ATL)],
                        acc.at[pl.ds(NS * APT, ATL)])


def _writeback_acc(acc, part_hbm, cid, sid):
    pltpu.sync_copy(acc.at[pl.ds(sid * APT, APT)],
                    part_hbm.at[pl.ds(cid * N_ATOMS + sid * APT, APT)])

    @pl.when(sid == NS - 1)
    def _():
        pltpu.sync_copy(acc.at[pl.ds(NS * APT, ATL)],
                        part_hbm.at[pl.ds(cid * N_ATOMS + NS * APT, ATL)])


@functools.partial(
    pl.kernel,
    out_type=jax.ShapeDtypeStruct((NC * N_ATOMS, HIDDEN), jnp.float32),
    mesh=_mesh,
    scratch_types=[
        pltpu.VMEM((EPW,), jnp.int32),
        pltpu.VMEM((CH,), jnp.int32),
        pltpu.VMEM((CH,), jnp.int32),
        pltpu.VMEM((CH, HIDDEN), jnp.float32),
        pltpu.VMEM((CH, HIDDEN), jnp.float32),
        pltpu.VMEM_SHARED((N_ATOMS, HIDDEN), jnp.float32),
        pltpu.SemaphoreType.DMA,
        pltpu.SemaphoreType.DMA,
        pltpu.SemaphoreType.DMA,
        pltpu.SemaphoreType.DMA,
    ],
)
def _sc_scatter(h_hbm, b2a_hbm, b2revb_hbm, zeros_hbm, part_hbm,
                rb, dv0, dv1, hv0, hv1, acc, ld0, ld1, sc0, sc1):
    cid = lax.axis_index("c")
    sid = lax.axis_index("s")
    wid = cid * NS + sid
    ebase = wid * EPW
    _zero_acc(zeros_hbm, acc, sid)
    pltpu.sync_copy(b2revb_hbm.at[pl.ds(ebase, EPW)], rb)
    plsc.subcore_barrier()

    def fire_load(c, hv, dv, ld):
        pltpu.async_copy(h_hbm.at[pl.ds(ebase + c * CH, CH)], hv, ld)
        pltpu.async_copy(b2a_hbm.at[rb.at[pl.ds(c * CH, CH)]], dv, ld)

    def wait_load(c, hv, dv, ld):
        pltpu.make_async_copy(h_hbm.at[pl.ds(ebase + c * CH, CH)], hv, ld).wait()
        pltpu.make_async_copy(b2a_hbm.at[rb.at[pl.ds(c * CH, CH)]], dv, ld).wait()

    def fire_scat(c, hv, dv, sc):
        pltpu.async_copy(hv, acc.at[dv], sc, add=True)

    def drain_scat(c, hv, dv, sc):
        pltpu.make_async_copy(hv, acc.at[dv], sc).wait()

    fire_load(0, hv0, dv0, ld0)

    def body(g, carry):
        c0 = 2 * g
        c1 = c0 + 1
        wait_load(c0, hv0, dv0, ld0)

        @pl.when(g > 0)
        def _():
            drain_scat(c0 - 1, hv1, dv1, sc1)
        fire_load(c1, hv1, dv1, ld1)
        fire_scat(c0, hv0, dv0, sc0)

        wait_load(c1, hv1, dv1, ld1)
        drain_scat(c0, hv0, dv0, sc0)

        @pl.when(c1 + 1 < NCHUNK)
        def _():
            fire_load(c1 + 1, hv0, dv0, ld0)
        fire_scat(c1, hv1, dv1, sc1)
        return carry

    lax.fori_loop(0, NPAIR, body, 0)
    drain_scat(NCHUNK - 1, hv1, dv1, sc1)
    plsc.subcore_barrier()
    _writeback_acc(acc, part_hbm, cid, sid)


@functools.partial(
    pl.kernel,
    out_type=(
        jax.ShapeDtypeStruct((N_EDGES, HIDDEN), jnp.float32),
        jax.ShapeDtypeStruct((NC * N_ATOMS, HIDDEN), jnp.float32),
    ),
    mesh=_mesh,
    scratch_types=[
        pltpu.VMEM((EPW,), jnp.int32),
        pltpu.VMEM((CH,), jnp.int32),
        pltpu.VMEM((CH,), jnp.int32),
        pltpu.VMEM((CH,), jnp.int32),
        pltpu.VMEM((CH,), jnp.int32),
        pltpu.VMEM((CH, HIDDEN), jnp.float32),
        pltpu.VMEM((CH, HIDDEN), jnp.float32),
        pltpu.VMEM((CH, HIDDEN), jnp.float32),
        pltpu.VMEM((CH, HIDDEN), jnp.float32),
        pltpu.VMEM((CH, HIDDEN), jnp.float32),
        pltpu.VMEM((CH, HIDDEN), jnp.float32),
        pltpu.VMEM_SHARED((N_ATOMS, HIDDEN), jnp.float32),
        pltpu.SemaphoreType.DMA,
        pltpu.SemaphoreType.DMA,
        pltpu.SemaphoreType.DMA,
        pltpu.SemaphoreType.DMA,
        pltpu.SemaphoreType.DMA,
        pltpu.SemaphoreType.DMA,
        pltpu.SemaphoreType.DMA,
        pltpu.SemaphoreType.DMA,
    ],
)
def _sc_update_scatter(h_hbm, z_hbm, y_hbm, b2a_hbm, b2revb_hbm,
                       zeros_hbm, hn_hbm, part_hbm,
                       rb, av0, av1, dv0, dv1, hv0, hv1, zv0, zv1,
                       yv0, yv1, acc, ld0, ld1, st0, st1, sc0, sc1,
                       avs0, avs1):
    cid = lax.axis_index("c")
    sid = lax.axis_index("s")
    wid = cid * NS + sid
    ebase = wid * EPW
    _zero_acc(zeros_hbm, acc, sid)
    pltpu.sync_copy(b2revb_hbm.at[pl.ds(ebase, EPW)], rb)
    plsc.subcore_barrier()

    def fire_av(c, av, avs):
        pltpu.async_copy(b2a_hbm.at[pl.ds(ebase + c * CH, CH)], av, avs)

    def wait_av(c, av, avs):
        pltpu.make_async_copy(b2a_hbm.at[pl.ds(ebase + c * CH, CH)], av, avs).wait()

    def fire_loads(c, av, dv, hv, zv, yv, ld):
        off = ebase + c * CH
        pltpu.async_copy(h_hbm.at[pl.ds(off, CH)], hv, ld)
        pltpu.async_copy(z_hbm.at[av], zv, ld)
        pltpu.async_copy(y_hbm.at[rb.at[pl.ds(c * CH, CH)]], yv, ld)
        pltpu.async_copy(b2a_hbm.at[rb.at[pl.ds(c * CH, CH)]], dv, ld)

    def wait_loads(c, av, dv, hv, zv, yv, ld):
        off = ebase + c * CH
        pltpu.make_async_copy(h_hbm.at[pl.ds(off, CH)], hv, ld).wait()
        pltpu.make_async_copy(z_hbm.at[av], zv, ld).wait()
        pltpu.make_async_copy(y_hbm.at[rb.at[pl.ds(c * CH, CH)]], yv, ld).wait()
        pltpu.make_async_copy(b2a_hbm.at[rb.at[pl.ds(c * CH, CH)]], dv, ld).wait()

    def compute(hv, zv, yv):
        def row(rr, c2):
            for j in range(HL):
                sl = pl.ds(j * 16, 16)
                hv[rr, sl] = jnp.maximum(
                    hv[rr, sl] + zv[rr, sl] - yv[rr, sl], 0.0)
            return c2
        lax.fori_loop(0, CH, row, 0)

    def fire_out(c, hv, dv, st, sc):
        pltpu.async_copy(hv, hn_hbm.at[pl.ds(ebase + c * CH, CH)], st)
        pltpu.async_copy(hv, acc.at[dv], sc, add=True)

    def drain_out(c, hv, dv, st, sc):
        pltpu.make_async_copy(hv, hn_hbm.at[pl.ds(ebase + c * CH, CH)], st).wait()
        pltpu.make_async_copy(hv, acc.at[dv], sc).wait()

    fire_av(0, av0, avs0)
    fire_av(1, av1, avs1)
    wait_av(0, av0, avs0)
    fire_loads(0, av0, dv0, hv0, zv0, yv0, ld0)

    def body(g, carry):
        c0 = 2 * g
        c1 = c0 + 1
        wait_loads(c0, av0, dv0, hv0, zv0, yv0, ld0)

        @pl.when(g > 0)
        def _():
            drain_out(c0 - 1, hv1, dv1, st1, sc1)

        @pl.when(c0 + 2 < NCHUNK)
        def _():
            fire_av(c0 + 2, av0, avs0)
        wait_av(c1, av1, avs1)
        fire_loads(c1, av1, dv1, hv1, zv1, yv1, ld1)
        compute(hv0, zv0, yv0)
        fire_out(c0, hv0, dv0, st0, sc0)

        wait_loads(c1, av1, dv1, hv1, zv1, yv1, ld1)
        drain_out(c0, hv0, dv0, st0, sc0)

        @pl.when(c1 + 2 < NCHUNK)
        def _():
            fire_av(c1 + 2, av1, avs1)

        @pl.when(c1 + 1 < NCHUNK)
        def _():
            wait_av(c1 + 1, av0, avs0)
            fire_loads(c1 + 1, av0, dv0, hv0, zv0, yv0, ld0)
        compute(hv1, zv1, yv1)
        fire_out(c1, hv1, dv1, st1, sc1)
        return carry

    lax.fori_loop(0, NPAIR, body, 0)
    drain_out(NCHUNK - 1, hv1, dv1, st1, sc1)
    plsc.subcore_barrier()
    _writeback_acc(acc, part_hbm, cid, sid)


BR = 1000
BA = 1000


def _tc_in_body(fb, w, b, o):
    o[...] = jnp.maximum(
        jnp.dot(fb[...].astype(jnp.bfloat16), w[...],
                preferred_element_type=jnp.float32) + b[...], 0.0)


def _tc_yz_body(x, wbf, pa, pb, w, b, y, z):
    y[...] = jnp.dot(x[...].astype(jnp.bfloat16), wbf[...],
                     preferred_element_type=jnp.float32)

    @pl.when(pl.program_id(0) >= N_EDGES // BR)
    def _():
        z[...] = jnp.dot(pa[...] + pb[...], w[...],
                         preferred_element_type=jnp.float32) + b[...]


def _tc_atom_body(pa, pb, fa, wt, wb, bb, nw, nb, ha, npred):
    h_atom = jnp.maximum(
        jnp.dot(pa[...] + pb[...], wt[...], preferred_element_type=jnp.float32)
        + jnp.dot(fa[...], wb[...], preferred_element_type=jnp.float32)
        + bb[...], 0.0)
    ha[...] = h_atom
    npred[...] = jnp.dot(h_atom, nw[...], preferred_element_type=jnp.float32) + nb[...]


def _tc_edge_body(h, w, b, o):
    o[...] = jnp.dot(h[...].astype(jnp.bfloat16), w[...],
                     preferred_element_type=jnp.float32) + b[...]


def _full(shape):
    return pl.BlockSpec(shape, lambda i: (0, 0))


def kernel(f_atoms, f_bonds, a2b, b2a, b2revb,
           W_in_w, W_in_b, W_msg_w, W_msg_b,
           W_atom_w, W_atom_b, node_w, node_b, edge_w, edge_b):
    del a2b
    FB = f_bonds.shape[1]
    zeros_a = jnp.zeros((N_ATOMS, HIDDEN), jnp.float32)
    b2a = b2a.astype(jnp.int32)
    b2revb = b2revb.astype(jnp.int32)

    h = pl.pallas_call(
        _tc_in_body,
        grid=(N_EDGES // BR,),
        in_specs=[pl.BlockSpec((BR, FB), lambda i: (i, 0)),
                  _full((FB, HIDDEN)), _full((1, HIDDEN))],
        out_specs=pl.BlockSpec((BR, HIDDEN), lambda i: (i, 0)),
        out_shape=jax.ShapeDtypeStruct((N_EDGES, HIDDEN), jnp.float32),
    )(f_bonds, W_in_w.astype(jnp.bfloat16), W_in_b.reshape(1, HIDDEN))

    NE_B = N_EDGES // BR
    yz_call = pl.pallas_call(
        _tc_yz_body,
        grid=(NE_B + N_ATOMS // BA,),
        in_specs=[pl.BlockSpec((BR, HIDDEN),
                               lambda i: (jnp.minimum(i, NE_B - 1), 0)),
                  _full((HIDDEN, HIDDEN)),
                  pl.BlockSpec((BA, HIDDEN),
                               lambda i: (jnp.maximum(i - NE_B, 0), 0)),
                  pl.BlockSpec((BA, HIDDEN),
                               lambda i: (jnp.maximum(i - NE_B, 0)
                                          + N_ATOMS // BA, 0)),
                  _full((HIDDEN, HIDDEN)), _full((1, HIDDEN))],
        out_specs=[pl.BlockSpec((BR, HIDDEN),
                                lambda i: (jnp.minimum(i, NE_B - 1), 0)),
                   pl.BlockSpec((BA, HIDDEN),
                                lambda i: (jnp.maximum(i - NE_B, 0), 0))],
        out_shape=[jax.ShapeDtypeStruct((N_EDGES, HIDDEN), jnp.float32),
                   jax.ShapeDtypeStruct((N_ATOMS, HIDDEN), jnp.float32)],
    )

    W_msg_bf = W_msg_w.astype(jnp.bfloat16)
    msg_b = W_msg_b.reshape(1, HIDDEN)
    part = _sc_scatter(h, b2a, b2revb, zeros_a)
    for _ in range(STEPS):
        y, z = yz_call(h, W_msg_bf, part, part, W_msg_w, msg_b)
        h, part = _sc_update_scatter(h, z, y, b2a, b2revb, zeros_a)

    h_atom, node_pred = pl.pallas_call(
        _tc_atom_body,
        grid=(N_ATOMS // BA,),
        in_specs=[pl.BlockSpec((BA, HIDDEN), lambda i: (i, 0)),
                  pl.BlockSpec((BA, HIDDEN), lambda i: (i + N_ATOMS // BA, 0)),
                  pl.BlockSpec((BA, f_atoms.shape[1]), lambda i: (i, 0)),
                  _full((HIDDEN, HIDDEN)), _full((f_atoms.shape[1], HIDDEN)),
                  _full((1, HIDDEN)),
                  _full((HIDDEN, node_w.shape[1])), _full((1, node_w.shape[1]))],
        out_specs=[pl.BlockSpec((BA, HIDDEN), lambda i: (i, 0)),
                   pl.BlockSpec((BA, node_w.shape[1]), lambda i: (i, 0))],
        out_shape=[jax.ShapeDtypeStruct((N_ATOMS, HIDDEN), jnp.float32),
                   jax.ShapeDtypeStruct((N_ATOMS, node_w.shape[1]), jnp.float32)],
    )(part, part, f_atoms, W_atom_w[:HIDDEN], W_atom_w[HIDDEN:],
      W_atom_b.reshape(1, HIDDEN), node_w, node_b.reshape(1, -1))

    edge_pred = pl.pallas_call(
        _tc_edge_body,
        grid=(N_EDGES // BR,),
        in_specs=[pl.BlockSpec((BR, HIDDEN), lambda i: (i, 0)),
                  _full((HIDDEN, edge_w.shape[1])), _full((1, edge_w.shape[1]))],
        out_specs=pl.BlockSpec((BR, edge_w.shape[1]), lambda i: (i, 0)),
        out_shape=jax.ShapeDtypeStruct((N_EDGES, edge_w.shape[1]), jnp.float32),
    )(h, edge_w.astype(jnp.bfloat16), edge_b.reshape(1, -1))

    return (node_pred, edge_pred, h_atom)


# speedup vs baseline: 1.0747x; 1.0747x over previous
import functools

import jax
import jax.numpy as jnp
from jax import lax
from jax.experimental import pallas as pl
from jax.experimental.pallas import tpu as pltpu
from jax.experimental.pallas import tpu_sc as plsc

N_ATOMS = 10000
N_EDGES = 320000
HIDDEN = 128
STEPS = 3

NC, NS = 2, 16
NW = NC * NS
EPW = N_EDGES // NW
CH = 40
NCHUNK = EPW // CH
NPAIR = NCHUNK // 2
assert NCHUNK == 2 * NPAIR
APT = 624
ATL = N_ATOMS - NS * APT
HL = HIDDEN // 16

_mesh = plsc.VectorSubcoreMesh(core_axis_name="c", subcore_axis_name="s")


def _zero_acc(zeros_hbm, acc, sid):
    pltpu.sync_copy(zeros_hbm.at[pl.ds(sid * APT, APT)],
                    acc.at[pl.ds(sid * APT, APT)])

    @pl.when(sid == NS - 1)
    def _():
        pltpu.sync_copy(zeros_hbm.at[pl.ds(NS * APT, ATL)],
                        acc.at[pl.ds(NS * APT, ATL)])


def _writeback_acc(acc, part_hbm, cid, sid):
    pltpu.sync_copy(acc.at[pl.ds(sid * APT, APT)],
                    part_hbm.at[pl.ds(cid * N_ATOMS + sid * APT, APT)])

    @pl.when(sid == NS - 1)
    def _():
        pltpu.sync_copy(acc.at[pl.ds(NS * APT, ATL)],
                        part_hbm.at[pl.ds(cid * N_ATOMS + NS * APT, ATL)])


SCH = 80
SNCHUNK = EPW // SCH
SNPAIR = SNCHUNK // 2
assert SNCHUNK == 2 * SNPAIR + 1


@functools.partial(
    pl.kernel,
    out_type=jax.ShapeDtypeStruct((NC * N_ATOMS, HIDDEN), jnp.float32),
    mesh=_mesh,
    scratch_types=[
        pltpu.VMEM((EPW,), jnp.int32),
        pltpu.VMEM((SCH,), jnp.int32),
        pltpu.VMEM((SCH,), jnp.int32),
        pltpu.VMEM((SCH, HIDDEN), jnp.float32),
        pltpu.VMEM((SCH, HIDDEN), jnp.float32),
        pltpu.VMEM_SHARED((N_ATOMS, HIDDEN), jnp.float32),
        pltpu.SemaphoreType.DMA,
        pltpu.SemaphoreType.DMA,
        pltpu.SemaphoreType.DMA,
        pltpu.SemaphoreType.DMA,
    ],
)
def _sc_scatter(h_hbm, b2a_hbm, b2revb_hbm, zeros_hbm, part_hbm,
                rb, dv0, dv1, hv0, hv1, acc, ld0, ld1, sc0, sc1):
    cid = lax.axis_index("c")
    sid = lax.axis_index("s")
    wid = cid * NS + sid
    ebase = wid * EPW
    _zero_acc(zeros_hbm, acc, sid)
    pltpu.sync_copy(b2revb_hbm.at[pl.ds(ebase, EPW)], rb)
    plsc.subcore_barrier()

    def fire_load(c, hv, dv, ld):
        pltpu.async_copy(h_hbm.at[pl.ds(ebase + c * SCH, SCH)], hv, ld)
        pltpu.async_copy(b2a_hbm.at[rb.at[pl.ds(c * SCH, SCH)]], dv, ld)

    def wait_load(c, hv, dv, ld):
        pltpu.make_async_copy(h_hbm.at[pl.ds(ebase + c * SCH, SCH)], hv, ld).wait()
        pltpu.make_async_copy(b2a_hbm.at[rb.at[pl.ds(c * SCH, SCH)]], dv, ld).wait()

    def fire_scat(c, hv, dv, sc):
        pltpu.async_copy(hv, acc.at[dv], sc, add=True)

    def drain_scat(c, hv, dv, sc):
        pltpu.make_async_copy(hv, acc.at[dv], sc).wait()

    fire_load(0, hv0, dv0, ld0)

    def body(g, carry):
        c0 = 2 * g
        c1 = c0 + 1
        wait_load(c0, hv0, dv0, ld0)

        @pl.when(g > 0)
        def _():
            drain_scat(c0 - 1, hv1, dv1, sc1)
        fire_load(c1, hv1, dv1, ld1)
        fire_scat(c0, hv0, dv0, sc0)

        wait_load(c1, hv1, dv1, ld1)
        drain_scat(c0, hv0, dv0, sc0)
        fire_load(c1 + 1, hv0, dv0, ld0)
        fire_scat(c1, hv1, dv1, sc1)
        return carry

    lax.fori_loop(0, SNPAIR, body, 0)
    clast = SNCHUNK - 1
    wait_load(clast, hv0, dv0, ld0)
    drain_scat(clast - 1, hv1, dv1, sc1)
    fire_scat(clast, hv0, dv0, sc0)
    drain_scat(clast, hv0, dv0, sc0)
    plsc.subcore_barrier()
    _writeback_acc(acc, part_hbm, cid, sid)


@functools.partial(
    pl.kernel,
    out_type=(
        jax.ShapeDtypeStruct((N_EDGES, HIDDEN), jnp.float32),
        jax.ShapeDtypeStruct((NC * N_ATOMS, HIDDEN), jnp.float32),
    ),
    mesh=_mesh,
    scratch_types=[
        pltpu.VMEM((EPW,), jnp.int32),
        pltpu.VMEM((CH,), jnp.int32),
        pltpu.VMEM((CH,), jnp.int32),
        pltpu.VMEM((CH,), jnp.int32),
        pltpu.VMEM((CH,), jnp.int32),
        pltpu.VMEM((CH, HIDDEN), jnp.float32),
        pltpu.VMEM((CH, HIDDEN), jnp.float32),
        pltpu.VMEM((CH, HIDDEN), jnp.float32),
        pltpu.VMEM((CH, HIDDEN), jnp.float32),
        pltpu.VMEM((CH, HIDDEN), jnp.float32),
        pltpu.VMEM((CH, HIDDEN), jnp.float32),
        pltpu.VMEM_SHARED((N_ATOMS, HIDDEN), jnp.float32),
        pltpu.SemaphoreType.DMA,
        pltpu.SemaphoreType.DMA,
        pltpu.SemaphoreType.DMA,
        pltpu.SemaphoreType.DMA,
        pltpu.SemaphoreType.DMA,
        pltpu.SemaphoreType.DMA,
        pltpu.SemaphoreType.DMA,
        pltpu.SemaphoreType.DMA,
    ],
)
def _sc_update_scatter(h_hbm, z_hbm, y_hbm, b2a_hbm, b2revb_hbm,
                       zeros_hbm, hn_hbm, part_hbm,
                       rb, av0, av1, dv0, dv1, hv0, hv1, zv0, zv1,
                       yv0, yv1, acc, ld0, ld1, st0, st1, sc0, sc1,
                       avs0, avs1):
    cid = lax.axis_index("c")
    sid = lax.axis_index("s")
    wid = cid * NS + sid
    ebase = wid * EPW
    _zero_acc(zeros_hbm, acc, sid)
    pltpu.sync_copy(b2revb_hbm.at[pl.ds(ebase, EPW)], rb)
    plsc.subcore_barrier()

    def fire_av(c, av, avs):
        pltpu.async_copy(b2a_hbm.at[pl.ds(ebase + c * CH, CH)], av, avs)

    def wait_av(c, av, avs):
        pltpu.make_async_copy(b2a_hbm.at[pl.ds(ebase + c * CH, CH)], av, avs).wait()

    def fire_loads(c, av, dv, hv, zv, yv, ld):
        off = ebase + c * CH
        pltpu.async_copy(h_hbm.at[pl.ds(off, CH)], hv, ld)
        pltpu.async_copy(z_hbm.at[av], zv, ld)
        pltpu.async_copy(y_hbm.at[rb.at[pl.ds(c * CH, CH)]], yv, ld)
        pltpu.async_copy(b2a_hbm.at[rb.at[pl.ds(c * CH, CH)]], dv, ld)

    def wait_loads(c, av, dv, hv, zv, yv, ld):
        off = ebase + c * CH
        pltpu.make_async_copy(h_hbm.at[pl.ds(off, CH)], hv, ld).wait()
        pltpu.make_async_copy(z_hbm.at[av], zv, ld).wait()
        pltpu.make_async_copy(y_hbm.at[rb.at[pl.ds(c * CH, CH)]], yv, ld).wait()
        pltpu.make_async_copy(b2a_hbm.at[rb.at[pl.ds(c * CH, CH)]], dv, ld).wait()

    def compute(hv, zv, yv):
        def row(rr, c2):
            for j in range(HL):
                sl = pl.ds(j * 16, 16)
                hv[rr, sl] = jnp.maximum(
                    hv[rr, sl] + zv[rr, sl] - yv[rr, sl], 0.0)
            return c2
        lax.fori_loop(0, CH, row, 0)

    def fire_out(c, hv, dv, st, sc):
        pltpu.async_copy(hv, hn_hbm.at[pl.ds(ebase + c * CH, CH)], st)
        pltpu.async_copy(hv, acc.at[dv], sc, add=True)

    def drain_out(c, hv, dv, st, sc):
        pltpu.make_async_copy(hv, hn_hbm.at[pl.ds(ebase + c * CH, CH)], st).wait()
        pltpu.make_async_copy(hv, acc.at[dv], sc).wait()

    fire_av(0, av0, avs0)
    fire_av(1, av1, avs1)
    wait_av(0, av0, avs0)
    fire_loads(0, av0, dv0, hv0, zv0, yv0, ld0)

    def body(g, carry):
        c0 = 2 * g
        c1 = c0 + 1
        wait_loads(c0, av0, dv0, hv0, zv0, yv0, ld0)

        @pl.when(g > 0)
        def _():
            drain_out(c0 - 1, hv1, dv1, st1, sc1)

        @pl.when(c0 + 2 < NCHUNK)
        def _():
            fire_av(c0 + 2, av0, avs0)
        wait_av(c1, av1, avs1)
        fire_loads(c1, av1, dv1, hv1, zv1, yv1, ld1)
        compute(hv0, zv0, yv0)
        fire_out(c0, hv0, dv0, st0, sc0)

        wait_loads(c1, av1, dv1, hv1, zv1, yv1, ld1)
        drain_out(c0, hv0, dv0, st0, sc0)

        @pl.when(c1 + 2 < NCHUNK)
        def _():
            fire_av(c1 + 2, av1, avs1)

        @pl.when(c1 + 1 < NCHUNK)
        def _():
            wait_av(c1 + 1, av0, avs0)
            fire_loads(c1 + 1, av0, dv0, hv0, zv0, yv0, ld0)
        compute(hv1, zv1, yv1)
        fire_out(c1, hv1, dv1, st1, sc1)
        return carry

    lax.fori_loop(0, NPAIR, body, 0)
    drain_out(NCHUNK - 1, hv1, dv1, st1, sc1)
    plsc.subcore_barrier()
    _writeback_acc(acc, part_hbm, cid, sid)


BR = 1000
BA = 1000


def _tc_in_body(fb, w, b, o):
    o[...] = jnp.maximum(
        jnp.dot(fb[...].astype(jnp.bfloat16), w[...],
                preferred_element_type=jnp.float32) + b[...], 0.0)


def _tc_mm_body(x, w, o):
    o[...] = jnp.dot(x[...].astype(jnp.bfloat16), w[...],
                     preferred_element_type=jnp.float32)


def _tc_z_body(pa, pb, w, b, o):
    o[...] = jnp.dot(pa[...] + pb[...], w[...],
                     preferred_element_type=jnp.float32) + b[...]


def _tc_atom_body(pa, pb, fa, wt, wb, bb, nw, nb, ha, npred):
    h_atom = jnp.maximum(
        jnp.dot(pa[...] + pb[...], wt[...], preferred_element_type=jnp.float32)
        + jnp.dot(fa[...], wb[...], preferred_element_type=jnp.float32)
        + bb[...], 0.0)
    ha[...] = h_atom
    npred[...] = jnp.dot(h_atom, nw[...], preferred_element_type=jnp.float32) + nb[...]


def _tc_edge_body(h, w, b, o):
    o[...] = jnp.dot(h[...].astype(jnp.bfloat16), w[...],
                     preferred_element_type=jnp.float32) + b[...]


def _full(shape):
    return pl.BlockSpec(shape, lambda i: (0, 0))


def kernel(f_atoms, f_bonds, a2b, b2a, b2revb,
           W_in_w, W_in_b, W_msg_w, W_msg_b,
           W_atom_w, W_atom_b, node_w, node_b, edge_w, edge_b):
    del a2b
    FB = f_bonds.shape[1]
    zeros_a = jnp.zeros((N_ATOMS, HIDDEN), jnp.float32)
    b2a = b2a.astype(jnp.int32)
    b2revb = b2revb.astype(jnp.int32)

    h = pl.pallas_call(
        _tc_in_body,
        grid=(N_EDGES // BR,),
        in_specs=[pl.BlockSpec((BR, FB), lambda i: (i, 0)),
                  _full((FB, HIDDEN)), _full((1, HIDDEN))],
        out_specs=pl.BlockSpec((BR, HIDDEN), lambda i: (i, 0)),
        out_shape=jax.ShapeDtypeStruct((N_EDGES, HIDDEN), jnp.float32),
    )(f_bonds, W_in_w.astype(jnp.bfloat16), W_in_b.reshape(1, HIDDEN))

    mm_call = pl.pallas_call(
        _tc_mm_body,
        grid=(N_EDGES // BR,),
        in_specs=[pl.BlockSpec((BR, HIDDEN), lambda i: (i, 0)),
                  _full((HIDDEN, HIDDEN))],
        out_specs=pl.BlockSpec((BR, HIDDEN), lambda i: (i, 0)),
        out_shape=jax.ShapeDtypeStruct((N_EDGES, HIDDEN), jnp.float32),
    )

    z_call = pl.pallas_call(
        _tc_z_body,
        grid=(N_ATOMS // BA,),
        in_specs=[pl.BlockSpec((BA, HIDDEN), lambda i: (i, 0)),
                  pl.BlockSpec((BA, HIDDEN), lambda i: (i + N_ATOMS // BA, 0)),
                  _full((HIDDEN, HIDDEN)), _full((1, HIDDEN))],
        out_specs=pl.BlockSpec((BA, HIDDEN), lambda i: (i, 0)),
        out_shape=jax.ShapeDtypeStruct((N_ATOMS, HIDDEN), jnp.float32),
    )

    W_msg_bf = W_msg_w.astype(jnp.bfloat16)
    msg_b = W_msg_b.reshape(1, HIDDEN)
    part = _sc_scatter(h, b2a, b2revb, zeros_a)
    for _ in range(STEPS):
        y = mm_call(h, W_msg_bf)
        z = z_call(part, part, W_msg_w, msg_b)
        h, part = _sc_update_scatter(h, z, y, b2a, b2revb, zeros_a)

    h_atom, node_pred = pl.pallas_call(
        _tc_atom_body,
        grid=(N_ATOMS // BA,),
        in_specs=[pl.BlockSpec((BA, HIDDEN), lambda i: (i, 0)),
                  pl.BlockSpec((BA, HIDDEN), lambda i: (i + N_ATOMS // BA, 0)),
                  pl.BlockSpec((BA, f_atoms.shape[1]), lambda i: (i, 0)),
                  _full((HIDDEN, HIDDEN)), _full((f_atoms.shape[1], HIDDEN)),
                  _full((1, HIDDEN)),
                  _full((HIDDEN, node_w.shape[1])), _full((1, node_w.shape[1]))],
        out_specs=[pl.BlockSpec((BA, HIDDEN), lambda i: (i, 0)),
                   pl.BlockSpec((BA, node_w.shape[1]), lambda i: (i, 0))],
        out_shape=[jax.ShapeDtypeStruct((N_ATOMS, HIDDEN), jnp.float32),
                   jax.ShapeDtypeStruct((N_ATOMS, node_w.shape[1]), jnp.float32)],
    )(part, part, f_atoms, W_atom_w[:HIDDEN], W_atom_w[HIDDEN:],
      W_atom_b.reshape(1, HIDDEN), node_w, node_b.reshape(1, -1))

    edge_pred = pl.pallas_call(
        _tc_edge_body,
        grid=(N_EDGES // BR,),
        in_specs=[pl.BlockSpec((BR, HIDDEN), lambda i: (i, 0)),
                  _full((HIDDEN, edge_w.shape[1])), _full((1, edge_w.shape[1]))],
        out_specs=pl.BlockSpec((BR, edge_w.shape[1]), lambda i: (i, 0)),
        out_shape=jax.ShapeDtypeStruct((N_EDGES, edge_w.shape[1]), jnp.float32),
    )(h, edge_w.astype(jnp.bfloat16), edge_b.reshape(1, -1))

    return (node_pred, edge_pred, h_atom)


# speedup vs baseline: 1.2419x; 1.1556x over previous
import functools

import jax
import jax.numpy as jnp
from jax import lax
from jax.experimental import pallas as pl
from jax.experimental.pallas import tpu as pltpu
from jax.experimental.pallas import tpu_sc as plsc

N_ATOMS = 10000
N_EDGES = 320000
HIDDEN = 128
STEPS = 3

NC, NS = 2, 16
NW = NC * NS
EPW = N_EDGES // NW
CH = 40
NCHUNK = EPW // CH
NPAIR = NCHUNK // 2
assert NCHUNK == 2 * NPAIR
APT = 624
ATL = N_ATOMS - NS * APT
HL = HIDDEN // 16

_mesh = plsc.VectorSubcoreMesh(core_axis_name="c", subcore_axis_name="s")


def _zero_acc(zeros_hbm, acc, sid):
    pltpu.sync_copy(zeros_hbm.at[pl.ds(sid * APT, APT)],
                    acc.at[pl.ds(sid * APT, APT)])

    @pl.when(sid == NS - 1)
    def _():
        pltpu.sync_copy(zeros_hbm.at[pl.ds(NS * APT, ATL)],
                        acc.at[pl.ds(NS * APT, ATL)])


def _writeback_acc(acc, part_hbm, cid, sid):
    pltpu.sync_copy(acc.at[pl.ds(sid * APT, APT)],
                    part_hbm.at[pl.ds(cid * N_ATOMS + sid * APT, APT)])

    @pl.when(sid == NS - 1)
    def _():
        pltpu.sync_copy(acc.at[pl.ds(NS * APT, ATL)],
                        part_hbm.at[pl.ds(cid * N_ATOMS + NS * APT, ATL)])


SCH = 80
SNCHUNK = EPW // SCH
SNPAIR = SNCHUNK // 2
assert SNCHUNK == 2 * SNPAIR + 1


@functools.partial(
    pl.kernel,
    out_type=jax.ShapeDtypeStruct((NC * N_ATOMS, HIDDEN), jnp.float32),
    mesh=_mesh,
    scratch_types=[
        pltpu.VMEM((EPW,), jnp.int32),
        pltpu.VMEM((SCH,), jnp.int32),
        pltpu.VMEM((SCH,), jnp.int32),
        pltpu.VMEM((SCH, HIDDEN), jnp.float32),
        pltpu.VMEM((SCH, HIDDEN), jnp.float32),
        pltpu.VMEM_SHARED((N_ATOMS, HIDDEN), jnp.float32),
        pltpu.SemaphoreType.DMA,
        pltpu.SemaphoreType.DMA,
        pltpu.SemaphoreType.DMA,
        pltpu.SemaphoreType.DMA,
    ],
)
def _sc_scatter(h_hbm, b2a_hbm, b2revb_hbm, zeros_hbm, part_hbm,
                rb, dv0, dv1, hv0, hv1, acc, ld0, ld1, sc0, sc1):
    cid = lax.axis_index("c")
    sid = lax.axis_index("s")
    wid = cid * NS + sid
    ebase = wid * EPW
    _zero_acc(zeros_hbm, acc, sid)
    pltpu.sync_copy(b2revb_hbm.at[pl.ds(ebase, EPW)], rb)
    plsc.subcore_barrier()

    def fire_load(c, hv, dv, ld):
        pltpu.async_copy(h_hbm.at[pl.ds(ebase + c * SCH, SCH)], hv, ld)
        pltpu.async_copy(b2a_hbm.at[rb.at[pl.ds(c * SCH, SCH)]], dv, ld)

    def wait_load(c, hv, dv, ld):
        pltpu.make_async_copy(h_hbm.at[pl.ds(ebase + c * SCH, SCH)], hv, ld).wait()
        pltpu.make_async_copy(b2a_hbm.at[rb.at[pl.ds(c * SCH, SCH)]], dv, ld).wait()

    def fire_scat(c, hv, dv, sc):
        pltpu.async_copy(hv, acc.at[dv], sc, add=True)

    def drain_scat(c, hv, dv, sc):
        pltpu.make_async_copy(hv, acc.at[dv], sc).wait()

    fire_load(0, hv0, dv0, ld0)

    def body(g, carry):
        c0 = 2 * g
        c1 = c0 + 1
        wait_load(c0, hv0, dv0, ld0)

        @pl.when(g > 0)
        def _():
            drain_scat(c0 - 1, hv1, dv1, sc1)
        fire_load(c1, hv1, dv1, ld1)
        fire_scat(c0, hv0, dv0, sc0)

        wait_load(c1, hv1, dv1, ld1)
        drain_scat(c0, hv0, dv0, sc0)
        fire_load(c1 + 1, hv0, dv0, ld0)
        fire_scat(c1, hv1, dv1, sc1)
        return carry

    lax.fori_loop(0, SNPAIR, body, 0)
    clast = SNCHUNK - 1
    wait_load(clast, hv0, dv0, ld0)
    drain_scat(clast - 1, hv1, dv1, sc1)
    fire_scat(clast, hv0, dv0, sc0)
    drain_scat(clast, hv0, dv0, sc0)
    plsc.subcore_barrier()
    _writeback_acc(acc, part_hbm, cid, sid)


@functools.partial(
    pl.kernel,
    out_type=(
        jax.ShapeDtypeStruct((N_EDGES, HIDDEN), jnp.float32),
        jax.ShapeDtypeStruct((NC * N_ATOMS, HIDDEN), jnp.float32),
    ),
    mesh=_mesh,
    scratch_types=[
        pltpu.VMEM((EPW,), jnp.int32),
        pltpu.VMEM((CH,), jnp.int32),
        pltpu.VMEM((CH,), jnp.int32),
        pltpu.VMEM((CH,), jnp.int32),
        pltpu.VMEM((CH,), jnp.int32),
        pltpu.VMEM((CH, HIDDEN), jnp.float32),
        pltpu.VMEM((CH, HIDDEN), jnp.float32),
        pltpu.VMEM((CH, HIDDEN), jnp.float32),
        pltpu.VMEM((CH, HIDDEN), jnp.float32),
        pltpu.VMEM((CH, HIDDEN), jnp.float32),
        pltpu.VMEM((CH, HIDDEN), jnp.float32),
        pltpu.VMEM_SHARED((N_ATOMS, HIDDEN), jnp.float32),
        pltpu.SemaphoreType.DMA,
        pltpu.SemaphoreType.DMA,
        pltpu.SemaphoreType.DMA,
        pltpu.SemaphoreType.DMA,
        pltpu.SemaphoreType.DMA,
        pltpu.SemaphoreType.DMA,
        pltpu.SemaphoreType.DMA,
        pltpu.SemaphoreType.DMA,
    ],
)
def _sc_update_scatter(h_hbm, z_hbm, y_hbm, b2a_hbm, b2revb_hbm,
                       zeros_hbm, hn_hbm, part_hbm,
                       rb, av0, av1, dv0, dv1, hv0, hv1, zv0, zv1,
                       yv0, yv1, acc, ld0, ld1, st0, st1, sc0, sc1,
                       avs0, avs1):
    cid = lax.axis_index("c")
    sid = lax.axis_index("s")
    wid = cid * NS + sid
    ebase = wid * EPW
    _zero_acc(zeros_hbm, acc, sid)
    pltpu.sync_copy(b2revb_hbm.at[pl.ds(ebase, EPW)], rb)
    plsc.subcore_barrier()

    def fire_av(c, av, avs):
        pltpu.async_copy(b2a_hbm.at[pl.ds(ebase + c * CH, CH)], av, avs)

    def wait_av(c, av, avs):
        pltpu.make_async_copy(b2a_hbm.at[pl.ds(ebase + c * CH, CH)], av, avs).wait()

    def fire_loads(c, av, dv, hv, zv, yv, ld):
        off = ebase + c * CH
        pltpu.async_copy(h_hbm.at[pl.ds(off, CH)], hv, ld)
        pltpu.async_copy(z_hbm.at[av], zv, ld)
        pltpu.async_copy(y_hbm.at[rb.at[pl.ds(c * CH, CH)]], yv, ld)
        pltpu.async_copy(b2a_hbm.at[rb.at[pl.ds(c * CH, CH)]], dv, ld)

    def wait_loads(c, av, dv, hv, zv, yv, ld):
        off = ebase + c * CH
        pltpu.make_async_copy(h_hbm.at[pl.ds(off, CH)], hv, ld).wait()
        pltpu.make_async_copy(z_hbm.at[av], zv, ld).wait()
        pltpu.make_async_copy(y_hbm.at[rb.at[pl.ds(c * CH, CH)]], yv, ld).wait()
        pltpu.make_async_copy(b2a_hbm.at[rb.at[pl.ds(c * CH, CH)]], dv, ld).wait()

    def compute(hv, zv, yv):
        def row(rr, c2):
            for j in range(HL):
                sl = pl.ds(j * 16, 16)
                hv[rr, sl] = jnp.maximum(
                    hv[rr, sl] + zv[rr, sl] - yv[rr, sl], 0.0)
            return c2
        lax.fori_loop(0, CH, row, 0)

    def fire_out(c, hv, dv, st, sc):
        pltpu.async_copy(hv, hn_hbm.at[pl.ds(ebase + c * CH, CH)], st)
        pltpu.async_copy(hv, acc.at[dv], sc, add=True)

    def drain_out(c, hv, dv, st, sc):
        pltpu.make_async_copy(hv, hn_hbm.at[pl.ds(ebase + c * CH, CH)], st).wait()
        pltpu.make_async_copy(hv, acc.at[dv], sc).wait()

    fire_av(0, av0, avs0)
    fire_av(1, av1, avs1)
    wait_av(0, av0, avs0)
    fire_loads(0, av0, dv0, hv0, zv0, yv0, ld0)

    def body(g, carry):
        c0 = 2 * g
        c1 = c0 + 1
        wait_loads(c0, av0, dv0, hv0, zv0, yv0, ld0)

        @pl.when(g > 0)
        def _():
            drain_out(c0 - 1, hv1, dv1, st1, sc1)

        @pl.when(c0 + 2 < NCHUNK)
        def _():
            fire_av(c0 + 2, av0, avs0)
        wait_av(c1, av1, avs1)
        fire_loads(c1, av1, dv1, hv1, zv1, yv1, ld1)
        compute(hv0, zv0, yv0)
        fire_out(c0, hv0, dv0, st0, sc0)

        wait_loads(c1, av1, dv1, hv1, zv1, yv1, ld1)
        drain_out(c0, hv0, dv0, st0, sc0)

        @pl.when(c1 + 2 < NCHUNK)
        def _():
            fire_av(c1 + 2, av1, avs1)

        @pl.when(c1 + 1 < NCHUNK)
        def _():
            wait_av(c1 + 1, av0, avs0)
            fire_loads(c1 + 1, av0, dv0, hv0, zv0, yv0, ld0)
        compute(hv1, zv1, yv1)
        fire_out(c1, hv1, dv1, st1, sc1)
        return carry

    lax.fori_loop(0, NPAIR, body, 0)
    drain_out(NCHUNK - 1, hv1, dv1, st1, sc1)
    plsc.subcore_barrier()
    _writeback_acc(acc, part_hbm, cid, sid)


BR = 2000
BA = 1000


def _tc_in_body(fb, w, b, o):
    o[...] = jnp.maximum(
        jnp.dot(fb[...].astype(jnp.bfloat16), w[...],
                preferred_element_type=jnp.float32) + b[...], 0.0)


def _tc_mm_body(x, w, o):
    o[...] = jnp.dot(x[...].astype(jnp.bfloat16), w[...],
                     preferred_element_type=jnp.float32)


def _tc_z_body(pa, pb, w, b, o):
    o[...] = jnp.dot(pa[...] + pb[...], w[...],
                     preferred_element_type=jnp.float32) + b[...]


def _tc_atom_body(pa, pb, fa, wt, wb, bb, nw, nb, ha, npred):
    h_atom = jnp.maximum(
        jnp.dot(pa[...] + pb[...], wt[...], preferred_element_type=jnp.float32)
        + jnp.dot(fa[...], wb[...], preferred_element_type=jnp.float32)
        + bb[...], 0.0)
    ha[...] = h_atom
    npred[...] = jnp.dot(h_atom, nw[...], preferred_element_type=jnp.float32) + nb[...]


def _tc_edge_body(h, w, b, o):
    o[...] = jnp.dot(h[...].astype(jnp.bfloat16), w[...],
                     preferred_element_type=jnp.float32) + b[...]


def _full(shape):
    return pl.BlockSpec(shape, lambda i: (0, 0))


def kernel(f_atoms, f_bonds, a2b, b2a, b2revb,
           W_in_w, W_in_b, W_msg_w, W_msg_b,
           W_atom_w, W_atom_b, node_w, node_b, edge_w, edge_b):
    del a2b
    FB = f_bonds.shape[1]
    zeros_a = jnp.zeros((N_ATOMS, HIDDEN), jnp.float32)
    b2a = b2a.astype(jnp.int32)
    b2revb = b2revb.astype(jnp.int32)

    h = pl.pallas_call(
        _tc_in_body,
        grid=(N_EDGES // BR,),
        in_specs=[pl.BlockSpec((BR, FB), lambda i: (i, 0)),
                  _full((FB, HIDDEN)), _full((1, HIDDEN))],
        out_specs=pl.BlockSpec((BR, HIDDEN), lambda i: (i, 0)),
        out_shape=jax.ShapeDtypeStruct((N_EDGES, HIDDEN), jnp.float32),
    )(f_bonds, W_in_w.astype(jnp.bfloat16), W_in_b.reshape(1, HIDDEN))

    mm_call = pl.pallas_call(
        _tc_mm_body,
        grid=(N_EDGES // BR,),
        in_specs=[pl.BlockSpec((BR, HIDDEN), lambda i: (i, 0)),
                  _full((HIDDEN, HIDDEN))],
        out_specs=pl.BlockSpec((BR, HIDDEN), lambda i: (i, 0)),
        out_shape=jax.ShapeDtypeStruct((N_EDGES, HIDDEN), jnp.float32),
    )

    z_call = pl.pallas_call(
        _tc_z_body,
        grid=(N_ATOMS // BA,),
        in_specs=[pl.BlockSpec((BA, HIDDEN), lambda i: (i, 0)),
                  pl.BlockSpec((BA, HIDDEN), lambda i: (i + N_ATOMS // BA, 0)),
                  _full((HIDDEN, HIDDEN)), _full((1, HIDDEN))],
        out_specs=pl.BlockSpec((BA, HIDDEN), lambda i: (i, 0)),
        out_shape=jax.ShapeDtypeStruct((N_ATOMS, HIDDEN), jnp.float32),
    )

    W_msg_bf = W_msg_w.astype(jnp.bfloat16)
    msg_b = W_msg_b.reshape(1, HIDDEN)
    part = _sc_scatter(h, b2a, b2revb, zeros_a)
    for _ in range(STEPS):
        y = mm_call(h, W_msg_bf)
        z = z_call(part, part, W_msg_w, msg_b)
        h, part = _sc_update_scatter(h, z, y, b2a, b2revb, zeros_a)

    h_atom, node_pred = pl.pallas_call(
        _tc_atom_body,
        grid=(N_ATOMS // BA,),
        in_specs=[pl.BlockSpec((BA, HIDDEN), lambda i: (i, 0)),
                  pl.BlockSpec((BA, HIDDEN), lambda i: (i + N_ATOMS // BA, 0)),
                  pl.BlockSpec((BA, f_atoms.shape[1]), lambda i: (i, 0)),
                  _full((HIDDEN, HIDDEN)), _full((f_atoms.shape[1], HIDDEN)),
                  _full((1, HIDDEN)),
                  _full((HIDDEN, node_w.shape[1])), _full((1, node_w.shape[1]))],
        out_specs=[pl.BlockSpec((BA, HIDDEN), lambda i: (i, 0)),
                   pl.BlockSpec((BA, node_w.shape[1]), lambda i: (i, 0))],
        out_shape=[jax.ShapeDtypeStruct((N_ATOMS, HIDDEN), jnp.float32),
                   jax.ShapeDtypeStruct((N_ATOMS, node_w.shape[1]), jnp.float32)],
    )(part, part, f_atoms, W_atom_w[:HIDDEN], W_atom_w[HIDDEN:],
      W_atom_b.reshape(1, HIDDEN), node_w, node_b.reshape(1, -1))

    edge_pred = pl.pallas_call(
        _tc_edge_body,
        grid=(N_EDGES // BR,),
        in_specs=[pl.BlockSpec((BR, HIDDEN), lambda i: (i, 0)),
                  _full((HIDDEN, edge_w.shape[1])), _full((1, edge_w.shape[1]))],
        out_specs=pl.BlockSpec((BR, edge_w.shape[1]), lambda i: (i, 0)),
        out_shape=jax.ShapeDtypeStruct((N_EDGES, edge_w.shape[1]), jnp.float32),
    )(h, edge_w.astype(jnp.bfloat16), edge_b.reshape(1, -1))

    return (node_pred, edge_pred, h_atom)


# speedup vs baseline: 1.3450x; 1.0830x over previous
import functools

import jax
import jax.numpy as jnp
from jax import lax
from jax.experimental import pallas as pl
from jax.experimental.pallas import tpu as pltpu
from jax.experimental.pallas import tpu_sc as plsc

N_ATOMS = 10000
N_EDGES = 320000
HIDDEN = 128
STEPS = 3

NC, NS = 2, 16
NW = NC * NS
EPW = N_EDGES // NW
CH = 40
NCHUNK = EPW // CH
NPAIR = NCHUNK // 2
assert NCHUNK == 2 * NPAIR
APT = 624
ATL = N_ATOMS - NS * APT
HL = HIDDEN // 16

_mesh = plsc.VectorSubcoreMesh(core_axis_name="c", subcore_axis_name="s")


def _zero_acc(zeros_hbm, acc, sid):
    pltpu.sync_copy(zeros_hbm.at[pl.ds(sid * APT, APT)],
                    acc.at[pl.ds(sid * APT, APT)])

    @pl.when(sid == NS - 1)
    def _():
        pltpu.sync_copy(zeros_hbm.at[pl.ds(NS * APT, ATL)],
                        acc.at[pl.ds(NS * APT, ATL)])


def _writeback_acc(acc, part_hbm, cid, sid):
    pltpu.sync_copy(acc.at[pl.ds(sid * APT, APT)],
                    part_hbm.at[pl.ds(cid * N_ATOMS + sid * APT, APT)])

    @pl.when(sid == NS - 1)
    def _():
        pltpu.sync_copy(acc.at[pl.ds(NS * APT, ATL)],
                        part_hbm.at[pl.ds(cid * N_ATOMS + NS * APT, ATL)])


SCH = 80
SNCHUNK = EPW // SCH
SNPAIR = SNCHUNK // 2
assert SNCHUNK == 2 * SNPAIR + 1


@functools.partial(
    pl.kernel,
    out_type=jax.ShapeDtypeStruct((NC * N_ATOMS, HIDDEN), jnp.float32),
    mesh=_mesh,
    scratch_types=[
        pltpu.VMEM((EPW,), jnp.int32),
        pltpu.VMEM((SCH,), jnp.int32),
        pltpu.VMEM((SCH,), jnp.int32),
        pltpu.VMEM((SCH, HIDDEN), jnp.float32),
        pltpu.VMEM((SCH, HIDDEN), jnp.float32),
        pltpu.VMEM_SHARED((N_ATOMS, HIDDEN), jnp.float32),
        pltpu.SemaphoreType.DMA,
        pltpu.SemaphoreType.DMA,
        pltpu.SemaphoreType.DMA,
        pltpu.SemaphoreType.DMA,
    ],
)
def _sc_scatter(h_hbm, b2a_hbm, b2revb_hbm, zeros_hbm, part_hbm,
                rb, dv0, dv1, hv0, hv1, acc, ld0, ld1, sc0, sc1):
    cid = lax.axis_index("c")
    sid = lax.axis_index("s")
    wid = cid * NS + sid
    ebase = wid * EPW
    _zero_acc(zeros_hbm, acc, sid)
    pltpu.sync_copy(b2revb_hbm.at[pl.ds(ebase, EPW)], rb)
    plsc.subcore_barrier()

    def fire_load(c, hv, dv, ld):
        pltpu.async_copy(h_hbm.at[pl.ds(ebase + c * SCH, SCH)], hv, ld)
        pltpu.async_copy(b2a_hbm.at[rb.at[pl.ds(c * SCH, SCH)]], dv, ld)

    def wait_load(c, hv, dv, ld):
        pltpu.make_async_copy(h_hbm.at[pl.ds(ebase + c * SCH, SCH)], hv, ld).wait()
        pltpu.make_async_copy(b2a_hbm.at[rb.at[pl.ds(c * SCH, SCH)]], dv, ld).wait()

    def fire_scat(c, hv, dv, sc):
        pltpu.async_copy(hv, acc.at[dv], sc, add=True)

    def drain_scat(c, hv, dv, sc):
        pltpu.make_async_copy(hv, acc.at[dv], sc).wait()

    fire_load(0, hv0, dv0, ld0)

    def body(g, carry):
        c0 = 2 * g
        c1 = c0 + 1
        wait_load(c0, hv0, dv0, ld0)

        @pl.when(g > 0)
        def _():
            drain_scat(c0 - 1, hv1, dv1, sc1)
        fire_load(c1, hv1, dv1, ld1)
        fire_scat(c0, hv0, dv0, sc0)

        wait_load(c1, hv1, dv1, ld1)
        drain_scat(c0, hv0, dv0, sc0)
        fire_load(c1 + 1, hv0, dv0, ld0)
        fire_scat(c1, hv1, dv1, sc1)
        return carry

    lax.fori_loop(0, SNPAIR, body, 0)
    clast = SNCHUNK - 1
    wait_load(clast, hv0, dv0, ld0)
    drain_scat(clast - 1, hv1, dv1, sc1)
    fire_scat(clast, hv0, dv0, sc0)
    drain_scat(clast, hv0, dv0, sc0)
    plsc.subcore_barrier()
    _writeback_acc(acc, part_hbm, cid, sid)


@functools.partial(
    pl.kernel,
    out_type=(
        jax.ShapeDtypeStruct((N_EDGES, HIDDEN), jnp.float32),
        jax.ShapeDtypeStruct((NC * N_ATOMS, HIDDEN), jnp.float32),
    ),
    mesh=_mesh,
    scratch_types=[
        pltpu.VMEM((EPW,), jnp.int32),
        pltpu.VMEM((CH,), jnp.int32),
        pltpu.VMEM((CH,), jnp.int32),
        pltpu.VMEM((CH,), jnp.int32),
        pltpu.VMEM((CH,), jnp.int32),
        pltpu.VMEM((CH, HIDDEN), jnp.float32),
        pltpu.VMEM((CH, HIDDEN), jnp.float32),
        pltpu.VMEM((CH, HIDDEN), jnp.float32),
        pltpu.VMEM((CH, HIDDEN), jnp.float32),
        pltpu.VMEM((CH, HIDDEN), jnp.float32),
        pltpu.VMEM((CH, HIDDEN), jnp.float32),
        pltpu.VMEM_SHARED((N_ATOMS, HIDDEN), jnp.float32),
        pltpu.SemaphoreType.DMA,
        pltpu.SemaphoreType.DMA,
        pltpu.SemaphoreType.DMA,
        pltpu.SemaphoreType.DMA,
        pltpu.SemaphoreType.DMA,
        pltpu.SemaphoreType.DMA,
        pltpu.SemaphoreType.DMA,
        pltpu.SemaphoreType.DMA,
    ],
)
def _sc_update_scatter(h_hbm, z_hbm, y_hbm, b2a_hbm, b2revb_hbm,
                       zeros_hbm, hn_hbm, part_hbm,
                       rb, av0, av1, dv0, dv1, hv0, hv1, zv0, zv1,
                       yv0, yv1, acc, ld0, ld1, st0, st1, sc0, sc1,
                       avs0, avs1):
    cid = lax.axis_index("c")
    sid = lax.axis_index("s")
    wid = cid * NS + sid
    ebase = wid * EPW
    _zero_acc(zeros_hbm, acc, sid)
    pltpu.sync_copy(b2revb_hbm.at[pl.ds(ebase, EPW)], rb)
    plsc.subcore_barrier()

    def fire_av(c, av, avs):
        pltpu.async_copy(b2a_hbm.at[pl.ds(ebase + c * CH, CH)], av, avs)

    def wait_av(c, av, avs):
        pltpu.make_async_copy(b2a_hbm.at[pl.ds(ebase + c * CH, CH)], av, avs).wait()

    def fire_loads(c, av, dv, hv, zv, yv, ld):
        off = ebase + c * CH
        pltpu.async_copy(h_hbm.at[pl.ds(off, CH)], hv, ld)
        pltpu.async_copy(z_hbm.at[av], zv, ld)
        pltpu.async_copy(y_hbm.at[rb.at[pl.ds(c * CH, CH)]], yv, ld)
        pltpu.async_copy(b2a_hbm.at[rb.at[pl.ds(c * CH, CH)]], dv, ld)

    def wait_loads(c, av, dv, hv, zv, yv, ld):
        off = ebase + c * CH
        pltpu.make_async_copy(h_hbm.at[pl.ds(off, CH)], hv, ld).wait()
        pltpu.make_async_copy(z_hbm.at[av], zv, ld).wait()
        pltpu.make_async_copy(y_hbm.at[rb.at[pl.ds(c * CH, CH)]], yv, ld).wait()
        pltpu.make_async_copy(b2a_hbm.at[rb.at[pl.ds(c * CH, CH)]], dv, ld).wait()

    def compute(hv, zv, yv):
        def row(rr, c2):
            for j in range(HL):
                sl = pl.ds(j * 16, 16)
                hv[rr, sl] = jnp.maximum(
                    hv[rr, sl] + zv[rr, sl] - yv[rr, sl], 0.0)
            return c2
        lax.fori_loop(0, CH, row, 0)

    def fire_out(c, hv, dv, st, sc):
        pltpu.async_copy(hv, hn_hbm.at[pl.ds(ebase + c * CH, CH)], st)
        pltpu.async_copy(hv, acc.at[dv], sc, add=True)

    def drain_out(c, hv, dv, st, sc):
        pltpu.make_async_copy(hv, hn_hbm.at[pl.ds(ebase + c * CH, CH)], st).wait()
        pltpu.make_async_copy(hv, acc.at[dv], sc).wait()

    fire_av(0, av0, avs0)
    fire_av(1, av1, avs1)
    wait_av(0, av0, avs0)
    fire_loads(0, av0, dv0, hv0, zv0, yv0, ld0)

    def body(g, carry):
        c0 = 2 * g
        c1 = c0 + 1
        wait_loads(c0, av0, dv0, hv0, zv0, yv0, ld0)

        @pl.when(g > 0)
        def _():
            drain_out(c0 - 1, hv1, dv1, st1, sc1)

        @pl.when(c0 + 2 < NCHUNK)
        def _():
            fire_av(c0 + 2, av0, avs0)
        wait_av(c1, av1, avs1)
        fire_loads(c1, av1, dv1, hv1, zv1, yv1, ld1)
        compute(hv0, zv0, yv0)
        fire_out(c0, hv0, dv0, st0, sc0)

        wait_loads(c1, av1, dv1, hv1, zv1, yv1, ld1)
        drain_out(c0, hv0, dv0, st0, sc0)

        @pl.when(c1 + 2 < NCHUNK)
        def _():
            fire_av(c1 + 2, av1, avs1)

        @pl.when(c1 + 1 < NCHUNK)
        def _():
            wait_av(c1 + 1, av0, avs0)
            fire_loads(c1 + 1, av0, dv0, hv0, zv0, yv0, ld0)
        compute(hv1, zv1, yv1)
        fire_out(c1, hv1, dv1, st1, sc1)
        return carry

    lax.fori_loop(0, NPAIR, body, 0)
    drain_out(NCHUNK - 1, hv1, dv1, st1, sc1)
    plsc.subcore_barrier()
    _writeback_acc(acc, part_hbm, cid, sid)


BR = 4000
BA = 2000


def _tc_in_body(fb, w, b, o):
    o[...] = jnp.maximum(
        jnp.dot(fb[...].astype(jnp.bfloat16), w[...],
                preferred_element_type=jnp.float32) + b[...], 0.0)


def _tc_mm_body(x, w, o):
    o[...] = jnp.dot(x[...].astype(jnp.bfloat16), w[...],
                     preferred_element_type=jnp.float32)


def _tc_z_body(pa, pb, w, b, o):
    o[...] = jnp.dot(pa[...] + pb[...], w[...],
                     preferred_element_type=jnp.float32) + b[...]


def _tc_atom_body(pa, pb, fa, wt, wb, bb, nw, nb, ha, npred):
    h_atom = jnp.maximum(
        jnp.dot(pa[...] + pb[...], wt[...], preferred_element_type=jnp.float32)
        + jnp.dot(fa[...], wb[...], preferred_element_type=jnp.float32)
        + bb[...], 0.0)
    ha[...] = h_atom
    npred[...] = jnp.dot(h_atom, nw[...], preferred_element_type=jnp.float32) + nb[...]


def _tc_edge_body(h, w, b, o):
    o[...] = jnp.dot(h[...].astype(jnp.bfloat16), w[...],
                     preferred_element_type=jnp.float32) + b[...]


def _full(shape):
    return pl.BlockSpec(shape, lambda i: (0, 0))


def kernel(f_atoms, f_bonds, a2b, b2a, b2revb,
           W_in_w, W_in_b, W_msg_w, W_msg_b,
           W_atom_w, W_atom_b, node_w, node_b, edge_w, edge_b):
    del a2b
    FB = f_bonds.shape[1]
    zeros_a = jnp.zeros((N_ATOMS, HIDDEN), jnp.float32)
    b2a = b2a.astype(jnp.int32)
    b2revb = b2revb.astype(jnp.int32)

    h = pl.pallas_call(
        _tc_in_body,
        grid=(N_EDGES // BR,),
        in_specs=[pl.BlockSpec((BR, FB), lambda i: (i, 0)),
                  _full((FB, HIDDEN)), _full((1, HIDDEN))],
        out_specs=pl.BlockSpec((BR, HIDDEN), lambda i: (i, 0)),
        out_shape=jax.ShapeDtypeStruct((N_EDGES, HIDDEN), jnp.float32),
    )(f_bonds, W_in_w.astype(jnp.bfloat16), W_in_b.reshape(1, HIDDEN))

    mm_call = pl.pallas_call(
        _tc_mm_body,
        grid=(N_EDGES // BR,),
        in_specs=[pl.BlockSpec((BR, HIDDEN), lambda i: (i, 0)),
                  _full((HIDDEN, HIDDEN))],
        out_specs=pl.BlockSpec((BR, HIDDEN), lambda i: (i, 0)),
        out_shape=jax.ShapeDtypeStruct((N_EDGES, HIDDEN), jnp.float32),
    )

    z_call = pl.pallas_call(
        _tc_z_body,
        grid=(N_ATOMS // BA,),
        in_specs=[pl.BlockSpec((BA, HIDDEN), lambda i: (i, 0)),
                  pl.BlockSpec((BA, HIDDEN), lambda i: (i + N_ATOMS // BA, 0)),
                  _full((HIDDEN, HIDDEN)), _full((1, HIDDEN))],
        out_specs=pl.BlockSpec((BA, HIDDEN), lambda i: (i, 0)),
        out_shape=jax.ShapeDtypeStruct((N_ATOMS, HIDDEN), jnp.float32),
    )

    W_msg_bf = W_msg_w.astype(jnp.bfloat16)
    msg_b = W_msg_b.reshape(1, HIDDEN)
    part = _sc_scatter(h, b2a, b2revb, zeros_a)
    for _ in range(STEPS):
        y = mm_call(h, W_msg_bf)
        z = z_call(part, part, W_msg_w, msg_b)
        h, part = _sc_update_scatter(h, z, y, b2a, b2revb, zeros_a)

    h_atom, node_pred = pl.pallas_call(
        _tc_atom_body,
        grid=(N_ATOMS // BA,),
        in_specs=[pl.BlockSpec((BA, HIDDEN), lambda i: (i, 0)),
                  pl.BlockSpec((BA, HIDDEN), lambda i: (i + N_ATOMS // BA, 0)),
                  pl.BlockSpec((BA, f_atoms.shape[1]), lambda i: (i, 0)),
                  _full((HIDDEN, HIDDEN)), _full((f_atoms.shape[1], HIDDEN)),
                  _full((1, HIDDEN)),
                  _full((HIDDEN, node_w.shape[1])), _full((1, node_w.shape[1]))],
        out_specs=[pl.BlockSpec((BA, HIDDEN), lambda i: (i, 0)),
                   pl.BlockSpec((BA, node_w.shape[1]), lambda i: (i, 0))],
        out_shape=[jax.ShapeDtypeStruct((N_ATOMS, HIDDEN), jnp.float32),
                   jax.ShapeDtypeStruct((N_ATOMS, node_w.shape[1]), jnp.float32)],
    )(part, part, f_atoms, W_atom_w[:HIDDEN], W_atom_w[HIDDEN:],
      W_atom_b.reshape(1, HIDDEN), node_w, node_b.reshape(1, -1))

    edge_pred = pl.pallas_call(
        _tc_edge_body,
        grid=(N_EDGES // BR,),
        in_specs=[pl.BlockSpec((BR, HIDDEN), lambda i: (i, 0)),
                  _full((HIDDEN, edge_w.shape[1])), _full((1, edge_w.shape[1]))],
        out_specs=pl.BlockSpec((BR, edge_w.shape[1]), lambda i: (i, 0)),
        out_shape=jax.ShapeDtypeStruct((N_EDGES, edge_w.shape[1]), jnp.float32),
    )(h, edge_w.astype(jnp.bfloat16), edge_b.reshape(1, -1))

    return (node_pred, edge_pred, h_atom)


# speedup vs baseline: 1.3702x; 1.0187x over previous
import functools

import jax
import jax.numpy as jnp
from jax import lax
from jax.experimental import pallas as pl
from jax.experimental.pallas import tpu as pltpu
from jax.experimental.pallas import tpu_sc as plsc

N_ATOMS = 10000
N_EDGES = 320000
HIDDEN = 128
STEPS = 3

NC, NS = 2, 16
NW = NC * NS
EPW = N_EDGES // NW
CH = 40
NCHUNK = EPW // CH
NPAIR = NCHUNK // 2
assert NCHUNK == 2 * NPAIR
APT = 624
ATL = N_ATOMS - NS * APT
HL = HIDDEN // 16

_mesh = plsc.VectorSubcoreMesh(core_axis_name="c", subcore_axis_name="s")


def _zero_acc(zeros_hbm, acc, sid):
    pltpu.sync_copy(zeros_hbm.at[pl.ds(sid * APT, APT)],
                    acc.at[pl.ds(sid * APT, APT)])

    @pl.when(sid == NS - 1)
    def _():
        pltpu.sync_copy(zeros_hbm.at[pl.ds(NS * APT, ATL)],
                        acc.at[pl.ds(NS * APT, ATL)])


def _writeback_acc(acc, part_hbm, cid, sid):
    pltpu.sync_copy(acc.at[pl.ds(sid * APT, APT)],
                    part_hbm.at[pl.ds(cid * N_ATOMS + sid * APT, APT)])

    @pl.when(sid == NS - 1)
    def _():
        pltpu.sync_copy(acc.at[pl.ds(NS * APT, ATL)],
                        part_hbm.at[pl.ds(cid * N_ATOMS + NS * APT, ATL)])


SCH = 80
SNCHUNK = EPW // SCH
SNPAIR = SNCHUNK // 2
assert SNCHUNK == 2 * SNPAIR + 1


@functools.partial(
    pl.kernel,
    out_type=jax.ShapeDtypeStruct((NC * N_ATOMS, HIDDEN), jnp.float32),
    mesh=_mesh,
    scratch_types=[
        pltpu.VMEM((EPW,), jnp.int32),
        pltpu.VMEM((SCH,), jnp.int32),
        pltpu.VMEM((SCH,), jnp.int32),
        pltpu.VMEM((SCH, HIDDEN), jnp.float32),
        pltpu.VMEM((SCH, HIDDEN), jnp.float32),
        pltpu.VMEM_SHARED((N_ATOMS, HIDDEN), jnp.float32),
        pltpu.SemaphoreType.DMA,
        pltpu.SemaphoreType.DMA,
        pltpu.SemaphoreType.DMA,
        pltpu.SemaphoreType.DMA,
    ],
)
def _sc_scatter(h_hbm, b2a_hbm, b2revb_hbm, zeros_hbm, part_hbm,
                rb, dv0, dv1, hv0, hv1, acc, ld0, ld1, sc0, sc1):
    cid = lax.axis_index("c")
    sid = lax.axis_index("s")
    wid = cid * NS + sid
    ebase = wid * EPW
    _zero_acc(zeros_hbm, acc, sid)
    pltpu.sync_copy(b2revb_hbm.at[pl.ds(ebase, EPW)], rb)
    plsc.subcore_barrier()

    def fire_load(c, hv, dv, ld):
        pltpu.async_copy(h_hbm.at[pl.ds(ebase + c * SCH, SCH)], hv, ld)
        pltpu.async_copy(b2a_hbm.at[rb.at[pl.ds(c * SCH, SCH)]], dv, ld)

    def wait_load(c, hv, dv, ld):
        pltpu.make_async_copy(h_hbm.at[pl.ds(ebase + c * SCH, SCH)], hv, ld).wait()
        pltpu.make_async_copy(b2a_hbm.at[rb.at[pl.ds(c * SCH, SCH)]], dv, ld).wait()

    def fire_scat(c, hv, dv, sc):
        pltpu.async_copy(hv, acc.at[dv], sc, add=True)

    def drain_scat(c, hv, dv, sc):
        pltpu.make_async_copy(hv, acc.at[dv], sc).wait()

    fire_load(0, hv0, dv0, ld0)

    def body(g, carry):
        c0 = 2 * g
        c1 = c0 + 1
        wait_load(c0, hv0, dv0, ld0)

        @pl.when(g > 0)
        def _():
            drain_scat(c0 - 1, hv1, dv1, sc1)
        fire_load(c1, hv1, dv1, ld1)
        fire_scat(c0, hv0, dv0, sc0)

        wait_load(c1, hv1, dv1, ld1)
        drain_scat(c0, hv0, dv0, sc0)
        fire_load(c1 + 1, hv0, dv0, ld0)
        fire_scat(c1, hv1, dv1, sc1)
        return carry

    lax.fori_loop(0, SNPAIR, body, 0)
    clast = SNCHUNK - 1
    wait_load(clast, hv0, dv0, ld0)
    drain_scat(clast - 1, hv1, dv1, sc1)
    fire_scat(clast, hv0, dv0, sc0)
    drain_scat(clast, hv0, dv0, sc0)
    plsc.subcore_barrier()
    _writeback_acc(acc, part_hbm, cid, sid)


@functools.partial(
    pl.kernel,
    out_type=(
        jax.ShapeDtypeStruct((N_EDGES, HIDDEN), jnp.float32),
        jax.ShapeDtypeStruct((NC * N_ATOMS, HIDDEN), jnp.float32),
    ),
    mesh=_mesh,
    scratch_types=[
        pltpu.VMEM((EPW,), jnp.int32),
        pltpu.VMEM((CH,), jnp.int32),
        pltpu.VMEM((CH,), jnp.int32),
        pltpu.VMEM((CH,), jnp.int32),
        pltpu.VMEM((CH,), jnp.int32),
        pltpu.VMEM((CH, HIDDEN), jnp.float32),
        pltpu.VMEM((CH, HIDDEN), jnp.float32),
        pltpu.VMEM((CH, HIDDEN), jnp.float32),
        pltpu.VMEM((CH, HIDDEN), jnp.float32),
        pltpu.VMEM((CH, HIDDEN), jnp.float32),
        pltpu.VMEM((CH, HIDDEN), jnp.float32),
        pltpu.VMEM_SHARED((N_ATOMS, HIDDEN), jnp.float32),
        pltpu.SemaphoreType.DMA,
        pltpu.SemaphoreType.DMA,
        pltpu.SemaphoreType.DMA,
        pltpu.SemaphoreType.DMA,
        pltpu.SemaphoreType.DMA,
        pltpu.SemaphoreType.DMA,
        pltpu.SemaphoreType.DMA,
        pltpu.SemaphoreType.DMA,
    ],
)
def _sc_update_scatter(h_hbm, z_hbm, y_hbm, b2a_hbm, b2revb_hbm,
                       zeros_hbm, hn_hbm, part_hbm,
                       rb, av0, av1, dv0, dv1, hv0, hv1, zv0, zv1,
                       yv0, yv1, acc, ld0, ld1, st0, st1, sc0, sc1,
                       avs0, avs1):
    cid = lax.axis_index("c")
    sid = lax.axis_index("s")
    wid = cid * NS + sid
    ebase = wid * EPW
    _zero_acc(zeros_hbm, acc, sid)
    pltpu.sync_copy(b2revb_hbm.at[pl.ds(ebase, EPW)], rb)
    plsc.subcore_barrier()

    def fire_av(c, av, avs):
        pltpu.async_copy(b2a_hbm.at[pl.ds(ebase + c * CH, CH)], av, avs)

    def wait_av(c, av, avs):
        pltpu.make_async_copy(b2a_hbm.at[pl.ds(ebase + c * CH, CH)], av, avs).wait()

    def fire_loads(c, av, dv, hv, zv, yv, ld):
        off = ebase + c * CH
        pltpu.async_copy(h_hbm.at[pl.ds(off, CH)], hv, ld)
        pltpu.async_copy(z_hbm.at[av], zv, ld)
        pltpu.async_copy(y_hbm.at[rb.at[pl.ds(c * CH, CH)]], yv, ld)
        pltpu.async_copy(b2a_hbm.at[rb.at[pl.ds(c * CH, CH)]], dv, ld)

    def wait_loads(c, av, dv, hv, zv, yv, ld):
        off = ebase + c * CH
        pltpu.make_async_copy(h_hbm.at[pl.ds(off, CH)], hv, ld).wait()
        pltpu.make_async_copy(z_hbm.at[av], zv, ld).wait()
        pltpu.make_async_copy(y_hbm.at[rb.at[pl.ds(c * CH, CH)]], yv, ld).wait()
        pltpu.make_async_copy(b2a_hbm.at[rb.at[pl.ds(c * CH, CH)]], dv, ld).wait()

    def compute(hv, zv, yv):
        def row(rr, c2):
            for j in range(HL):
                sl = pl.ds(j * 16, 16)
                hv[rr, sl] = jnp.maximum(
                    hv[rr, sl] + zv[rr, sl] - yv[rr, sl], 0.0)
            return c2
        lax.fori_loop(0, CH, row, 0)

    def fire_out(c, hv, dv, st, sc):
        pltpu.async_copy(hv, hn_hbm.at[pl.ds(ebase + c * CH, CH)], st)
        pltpu.async_copy(hv, acc.at[dv], sc, add=True)

    def drain_out(c, hv, dv, st, sc):
        pltpu.make_async_copy(hv, hn_hbm.at[pl.ds(ebase + c * CH, CH)], st).wait()
        pltpu.make_async_copy(hv, acc.at[dv], sc).wait()

    fire_av(0, av0, avs0)
    fire_av(1, av1, avs1)
    wait_av(0, av0, avs0)
    fire_loads(0, av0, dv0, hv0, zv0, yv0, ld0)

    def body(g, carry):
        c0 = 2 * g
        c1 = c0 + 1
        wait_loads(c0, av0, dv0, hv0, zv0, yv0, ld0)

        @pl.when(g > 0)
        def _():
            drain_out(c0 - 1, hv1, dv1, st1, sc1)

        @pl.when(c0 + 2 < NCHUNK)
        def _():
            fire_av(c0 + 2, av0, avs0)
        wait_av(c1, av1, avs1)
        fire_loads(c1, av1, dv1, hv1, zv1, yv1, ld1)
        compute(hv0, zv0, yv0)
        fire_out(c0, hv0, dv0, st0, sc0)

        wait_loads(c1, av1, dv1, hv1, zv1, yv1, ld1)
        drain_out(c0, hv0, dv0, st0, sc0)

        @pl.when(c1 + 2 < NCHUNK)
        def _():
            fire_av(c1 + 2, av1, avs1)

        @pl.when(c1 + 1 < NCHUNK)
        def _():
            wait_av(c1 + 1, av0, avs0)
            fire_loads(c1 + 1, av0, dv0, hv0, zv0, yv0, ld0)
        compute(hv1, zv1, yv1)
        fire_out(c1, hv1, dv1, st1, sc1)
        return carry

    lax.fori_loop(0, NPAIR, body, 0)
    drain_out(NCHUNK - 1, hv1, dv1, st1, sc1)
    plsc.subcore_barrier()
    _writeback_acc(acc, part_hbm, cid, sid)


BR = 8000
BA = 5000


def _tc_in_body(fb, w, b, o):
    o[...] = jnp.maximum(
        jnp.dot(fb[...].astype(jnp.bfloat16), w[...],
                preferred_element_type=jnp.float32) + b[...], 0.0)


def _tc_mm_body(x, w, o):
    o[...] = jnp.dot(x[...].astype(jnp.bfloat16), w[...],
                     preferred_element_type=jnp.float32)


def _tc_z_body(pa, pb, w, b, o):
    o[...] = jnp.dot(pa[...] + pb[...], w[...],
                     preferred_element_type=jnp.float32) + b[...]


def _tc_atom_body(pa, pb, fa, wt, wb, bb, nw, nb, ha, npred):
    h_atom = jnp.maximum(
        jnp.dot(pa[...] + pb[...], wt[...], preferred_element_type=jnp.float32)
        + jnp.dot(fa[...], wb[...], preferred_element_type=jnp.float32)
        + bb[...], 0.0)
    ha[...] = h_atom
    npred[...] = jnp.dot(h_atom, nw[...], preferred_element_type=jnp.float32) + nb[...]


def _tc_edge_body(h, w, b, o):
    o[...] = jnp.dot(h[...].astype(jnp.bfloat16), w[...],
                     preferred_element_type=jnp.float32) + b[...]


def _full(shape):
    return pl.BlockSpec(shape, lambda i: (0, 0))


def kernel(f_atoms, f_bonds, a2b, b2a, b2revb,
           W_in_w, W_in_b, W_msg_w, W_msg_b,
           W_atom_w, W_atom_b, node_w, node_b, edge_w, edge_b):
    del a2b
    FB = f_bonds.shape[1]
    zeros_a = jnp.zeros((N_ATOMS, HIDDEN), jnp.float32)
    b2a = b2a.astype(jnp.int32)
    b2revb = b2revb.astype(jnp.int32)

    h = pl.pallas_call(
        _tc_in_body,
        grid=(N_EDGES // BR,),
        in_specs=[pl.BlockSpec((BR, FB), lambda i: (i, 0)),
                  _full((FB, HIDDEN)), _full((1, HIDDEN))],
        out_specs=pl.BlockSpec((BR, HIDDEN), lambda i: (i, 0)),
        out_shape=jax.ShapeDtypeStruct((N_EDGES, HIDDEN), jnp.float32),
    )(f_bonds, W_in_w.astype(jnp.bfloat16), W_in_b.reshape(1, HIDDEN))

    mm_call = pl.pallas_call(
        _tc_mm_body,
        grid=(N_EDGES // BR,),
        in_specs=[pl.BlockSpec((BR, HIDDEN), lambda i: (i, 0)),
                  _full((HIDDEN, HIDDEN))],
        out_specs=pl.BlockSpec((BR, HIDDEN), lambda i: (i, 0)),
        out_shape=jax.ShapeDtypeStruct((N_EDGES, HIDDEN), jnp.float32),
    )

    z_call = pl.pallas_call(
        _tc_z_body,
        grid=(N_ATOMS // BA,),
        in_specs=[pl.BlockSpec((BA, HIDDEN), lambda i: (i, 0)),
                  pl.BlockSpec((BA, HIDDEN), lambda i: (i + N_ATOMS // BA, 0)),
                  _full((HIDDEN, HIDDEN)), _full((1, HIDDEN))],
        out_specs=pl.BlockSpec((BA, HIDDEN), lambda i: (i, 0)),
        out_shape=jax.ShapeDtypeStruct((N_ATOMS, HIDDEN), jnp.float32),
    )

    W_msg_bf = W_msg_w.astype(jnp.bfloat16)
    msg_b = W_msg_b.reshape(1, HIDDEN)
    part = _sc_scatter(h, b2a, b2revb, zeros_a)
    for _ in range(STEPS):
        y = mm_call(h, W_msg_bf)
        z = z_call(part, part, W_msg_w, msg_b)
        h, part = _sc_update_scatter(h, z, y, b2a, b2revb, zeros_a)

    h_atom, node_pred = pl.pallas_call(
        _tc_atom_body,
        grid=(N_ATOMS // BA,),
        in_specs=[pl.BlockSpec((BA, HIDDEN), lambda i: (i, 0)),
                  pl.BlockSpec((BA, HIDDEN), lambda i: (i + N_ATOMS // BA, 0)),
                  pl.BlockSpec((BA, f_atoms.shape[1]), lambda i: (i, 0)),
                  _full((HIDDEN, HIDDEN)), _full((f_atoms.shape[1], HIDDEN)),
                  _full((1, HIDDEN)),
                  _full((HIDDEN, node_w.shape[1])), _full((1, node_w.shape[1]))],
        out_specs=[pl.BlockSpec((BA, HIDDEN), lambda i: (i, 0)),
                   pl.BlockSpec((BA, node_w.shape[1]), lambda i: (i, 0))],
        out_shape=[jax.ShapeDtypeStruct((N_ATOMS, HIDDEN), jnp.float32),
                   jax.ShapeDtypeStruct((N_ATOMS, node_w.shape[1]), jnp.float32)],
    )(part, part, f_atoms, W_atom_w[:HIDDEN], W_atom_w[HIDDEN:],
      W_atom_b.reshape(1, HIDDEN), node_w, node_b.reshape(1, -1))

    edge_pred = pl.pallas_call(
        _tc_edge_body,
        grid=(N_EDGES // BR,),
        in_specs=[pl.BlockSpec((BR, HIDDEN), lambda i: (i, 0)),
                  _full((HIDDEN, edge_w.shape[1])), _full((1, edge_w.shape[1]))],
        out_specs=pl.BlockSpec((BR, edge_w.shape[1]), lambda i: (i, 0)),
        out_shape=jax.ShapeDtypeStruct((N_EDGES, edge_w.shape[1]), jnp.float32),
    )(h, edge_w.astype(jnp.bfloat16), edge_b.reshape(1, -1))

    return (node_pred, edge_pred, h_atom)


# speedup vs baseline: 1.3741x; 1.0028x over previous
import functools

import jax
import jax.numpy as jnp
from jax import lax
from jax.experimental import pallas as pl
from jax.experimental.pallas import tpu as pltpu
from jax.experimental.pallas import tpu_sc as plsc

N_ATOMS = 10000
N_EDGES = 320000
HIDDEN = 128
STEPS = 3

NC, NS = 2, 16
NW = NC * NS
EPW = N_EDGES // NW
CH = 40
NCHUNK = EPW // CH
NPAIR = NCHUNK // 2
assert NCHUNK == 2 * NPAIR
APT = 624
ATL = N_ATOMS - NS * APT
HL = HIDDEN // 16

_mesh = plsc.VectorSubcoreMesh(core_axis_name="c", subcore_axis_name="s")


def _zero_acc(zeros_hbm, acc, sid):
    pltpu.sync_copy(zeros_hbm.at[pl.ds(sid * APT, APT)],
                    acc.at[pl.ds(sid * APT, APT)])

    @pl.when(sid == NS - 1)
    def _():
        pltpu.sync_copy(zeros_hbm.at[pl.ds(NS * APT, ATL)],
                        acc.at[pl.ds(NS * APT, ATL)])


def _writeback_acc(acc, part_hbm, cid, sid):
    pltpu.sync_copy(acc.at[pl.ds(sid * APT, APT)],
                    part_hbm.at[pl.ds(cid * N_ATOMS + sid * APT, APT)])

    @pl.when(sid == NS - 1)
    def _():
        pltpu.sync_copy(acc.at[pl.ds(NS * APT, ATL)],
                        part_hbm.at[pl.ds(cid * N_ATOMS + NS * APT, ATL)])


SCH = 80
SNCHUNK = EPW // SCH
SNPAIR = SNCHUNK // 2
assert SNCHUNK == 2 * SNPAIR + 1


@functools.partial(
    pl.kernel,
    out_type=jax.ShapeDtypeStruct((NC * N_ATOMS, HIDDEN), jnp.float32),
    mesh=_mesh,
    scratch_types=[
        pltpu.VMEM((EPW,), jnp.int32),
        pltpu.VMEM((SCH,), jnp.int32),
        pltpu.VMEM((SCH,), jnp.int32),
        pltpu.VMEM((SCH, HIDDEN), jnp.float32),
        pltpu.VMEM((SCH, HIDDEN), jnp.float32),
        pltpu.VMEM_SHARED((N_ATOMS, HIDDEN), jnp.float32),
        pltpu.SemaphoreType.DMA,
        pltpu.SemaphoreType.DMA,
        pltpu.SemaphoreType.DMA,
        pltpu.SemaphoreType.DMA,
    ],
)
def _sc_scatter(h_hbm, b2a_hbm, b2revb_hbm, zeros_hbm, part_hbm,
                rb, dv0, dv1, hv0, hv1, acc, ld0, ld1, sc0, sc1):
    cid = lax.axis_index("c")
    sid = lax.axis_index("s")
    wid = cid * NS + sid
    ebase = wid * EPW
    _zero_acc(zeros_hbm, acc, sid)
    pltpu.sync_copy(b2revb_hbm.at[pl.ds(ebase, EPW)], rb)
    plsc.subcore_barrier()

    def fire_load(c, hv, dv, ld):
        pltpu.async_copy(h_hbm.at[pl.ds(ebase + c * SCH, SCH)], hv, ld)
        pltpu.async_copy(b2a_hbm.at[rb.at[pl.ds(c * SCH, SCH)]], dv, ld)

    def wait_load(c, hv, dv, ld):
        pltpu.make_async_copy(h_hbm.at[pl.ds(ebase + c * SCH, SCH)], hv, ld).wait()
        pltpu.make_async_copy(b2a_hbm.at[rb.at[pl.ds(c * SCH, SCH)]], dv, ld).wait()

    def fire_scat(c, hv, dv, sc):
        pltpu.async_copy(hv, acc.at[dv], sc, add=True)

    def drain_scat(c, hv, dv, sc):
        pltpu.make_async_copy(hv, acc.at[dv], sc).wait()

    fire_load(0, hv0, dv0, ld0)

    def body(g, carry):
        c0 = 2 * g
        c1 = c0 + 1
        wait_load(c0, hv0, dv0, ld0)

        @pl.when(g > 0)
        def _():
            drain_scat(c0 - 1, hv1, dv1, sc1)
        fire_load(c1, hv1, dv1, ld1)
        fire_scat(c0, hv0, dv0, sc0)

        wait_load(c1, hv1, dv1, ld1)
        drain_scat(c0, hv0, dv0, sc0)
        fire_load(c1 + 1, hv0, dv0, ld0)
        fire_scat(c1, hv1, dv1, sc1)
        return carry

    lax.fori_loop(0, SNPAIR, body, 0)
    clast = SNCHUNK - 1
    wait_load(clast, hv0, dv0, ld0)
    drain_scat(clast - 1, hv1, dv1, sc1)
    fire_scat(clast, hv0, dv0, sc0)
    drain_scat(clast, hv0, dv0, sc0)
    plsc.subcore_barrier()
    _writeback_acc(acc, part_hbm, cid, sid)


@functools.partial(
    pl.kernel,
    out_type=(
        jax.ShapeDtypeStruct((N_EDGES, HIDDEN), jnp.float32),
        jax.ShapeDtypeStruct((NC * N_ATOMS, HIDDEN), jnp.float32),
    ),
    mesh=_mesh,
    scratch_types=[
        pltpu.VMEM((EPW,), jnp.int32),
        pltpu.VMEM((CH,), jnp.int32),
        pltpu.VMEM((CH,), jnp.int32),
        pltpu.VMEM((CH,), jnp.int32),
        pltpu.VMEM((CH,), jnp.int32),
        pltpu.VMEM((CH, HIDDEN), jnp.float32),
        pltpu.VMEM((CH, HIDDEN), jnp.float32),
        pltpu.VMEM((CH, HIDDEN), jnp.float32),
        pltpu.VMEM((CH, HIDDEN), jnp.float32),
        pltpu.VMEM((CH, HIDDEN), jnp.float32),
        pltpu.VMEM((CH, HIDDEN), jnp.float32),
        pltpu.VMEM_SHARED((N_ATOMS, HIDDEN), jnp.float32),
        pltpu.SemaphoreType.DMA,
        pltpu.SemaphoreType.DMA,
        pltpu.SemaphoreType.DMA,
        pltpu.SemaphoreType.DMA,
        pltpu.SemaphoreType.DMA,
        pltpu.SemaphoreType.DMA,
        pltpu.SemaphoreType.DMA,
        pltpu.SemaphoreType.DMA,
    ],
)
def _sc_update_scatter(h_hbm, z_hbm, y_hbm, b2a_hbm, b2revb_hbm,
                       zeros_hbm, hn_hbm, part_hbm,
                       rb, av0, av1, dv0, dv1, hv0, hv1, zv0, zv1,
                       yv0, yv1, acc, ld0, ld1, st0, st1, sc0, sc1,
                       avs0, avs1):
    cid = lax.axis_index("c")
    sid = lax.axis_index("s")
    wid = cid * NS + sid
    ebase = wid * EPW
    _zero_acc(zeros_hbm, acc, sid)
    pltpu.sync_copy(b2revb_hbm.at[pl.ds(ebase, EPW)], rb)
    plsc.subcore_barrier()

    def fire_av(c, av, avs):
        pltpu.async_copy(b2a_hbm.at[pl.ds(ebase + c * CH, CH)], av, avs)

    def wait_av(c, av, avs):
        pltpu.make_async_copy(b2a_hbm.at[pl.ds(ebase + c * CH, CH)], av, avs).wait()

    def fire_loads(c, av, dv, hv, zv, yv, ld):
        off = ebase + c * CH
        pltpu.async_copy(h_hbm.at[pl.ds(off, CH)], hv, ld)
        pltpu.async_copy(z_hbm.at[av], zv, ld)
        pltpu.async_copy(y_hbm.at[rb.at[pl.ds(c * CH, CH)]], yv, ld)
        pltpu.async_copy(b2a_hbm.at[rb.at[pl.ds(c * CH, CH)]], dv, ld)

    def wait_loads(c, av, dv, hv, zv, yv, ld):
        off = ebase + c * CH
        pltpu.make_async_copy(h_hbm.at[pl.ds(off, CH)], hv, ld).wait()
        pltpu.make_async_copy(z_hbm.at[av], zv, ld).wait()
        pltpu.make_async_copy(y_hbm.at[rb.at[pl.ds(c * CH, CH)]], yv, ld).wait()
        pltpu.make_async_copy(b2a_hbm.at[rb.at[pl.ds(c * CH, CH)]], dv, ld).wait()

    def compute(hv, zv, yv):
        def row(rr, c2):
            for j in range(HL):
                sl = pl.ds(j * 16, 16)
                hv[rr, sl] = jnp.maximum(
                    hv[rr, sl] + zv[rr, sl] - yv[rr, sl], 0.0)
            return c2
        lax.fori_loop(0, CH, row, 0)

    def fire_out(c, hv, dv, st, sc):
        pltpu.async_copy(hv, hn_hbm.at[pl.ds(ebase + c * CH, CH)], st)
        pltpu.async_copy(hv, acc.at[dv], sc, add=True)

    def drain_out(c, hv, dv, st, sc):
        pltpu.make_async_copy(hv, hn_hbm.at[pl.ds(ebase + c * CH, CH)], st).wait()
        pltpu.make_async_copy(hv, acc.at[dv], sc).wait()

    fire_av(0, av0, avs0)
    fire_av(1, av1, avs1)
    wait_av(0, av0, avs0)
    fire_loads(0, av0, dv0, hv0, zv0, yv0, ld0)

    def body(g, carry):
        c0 = 2 * g
        c1 = c0 + 1
        wait_loads(c0, av0, dv0, hv0, zv0, yv0, ld0)

        @pl.when(g > 0)
        def _():
            drain_out(c0 - 1, hv1, dv1, st1, sc1)

        @pl.when(c0 + 2 < NCHUNK)
        def _():
            fire_av(c0 + 2, av0, avs0)
        wait_av(c1, av1, avs1)
        fire_loads(c1, av1, dv1, hv1, zv1, yv1, ld1)
        compute(hv0, zv0, yv0)
        fire_out(c0, hv0, dv0, st0, sc0)

        wait_loads(c1, av1, dv1, hv1, zv1, yv1, ld1)
        drain_out(c0, hv0, dv0, st0, sc0)

        @pl.when(c1 + 2 < NCHUNK)
        def _():
            fire_av(c1 + 2, av1, avs1)

        @pl.when(c1 + 1 < NCHUNK)
        def _():
            wait_av(c1 + 1, av0, avs0)
            fire_loads(c1 + 1, av0, dv0, hv0, zv0, yv0, ld0)
        compute(hv1, zv1, yv1)
        fire_out(c1, hv1, dv1, st1, sc1)
        return carry

    lax.fori_loop(0, NPAIR, body, 0)
    drain_out(NCHUNK - 1, hv1, dv1, st1, sc1)
    plsc.subcore_barrier()
    _writeback_acc(acc, part_hbm, cid, sid)


BR = 16000
BA = 5000


def _tc_in_body(fb, w, b, o):
    o[...] = jnp.maximum(
        jnp.dot(fb[...].astype(jnp.bfloat16), w[...],
                preferred_element_type=jnp.float32) + b[...], 0.0)


def _tc_mm_body(x, w, o):
    o[...] = jnp.dot(x[...].astype(jnp.bfloat16), w[...],
                     preferred_element_type=jnp.float32)


def _tc_z_body(pa, pb, w, b, o):
    o[...] = jnp.dot(pa[...] + pb[...], w[...],
                     preferred_element_type=jnp.float32) + b[...]


def _tc_atom_body(pa, pb, fa, wt, wb, bb, nw, nb, ha, npred):
    h_atom = jnp.maximum(
        jnp.dot(pa[...] + pb[...], wt[...], preferred_element_type=jnp.float32)
        + jnp.dot(fa[...], wb[...], preferred_element_type=jnp.float32)
        + bb[...], 0.0)
    ha[...] = h_atom
    npred[...] = jnp.dot(h_atom, nw[...], preferred_element_type=jnp.float32) + nb[...]


def _tc_edge_body(h, w, b, o):
    o[...] = jnp.dot(h[...].astype(jnp.bfloat16), w[...],
                     preferred_element_type=jnp.float32) + b[...]


def _full(shape):
    return pl.BlockSpec(shape, lambda i: (0, 0))


def kernel(f_atoms, f_bonds, a2b, b2a, b2revb,
           W_in_w, W_in_b, W_msg_w, W_msg_b,
           W_atom_w, W_atom_b, node_w, node_b, edge_w, edge_b):
    del a2b
    FB = f_bonds.shape[1]
    zeros_a = jnp.zeros((N_ATOMS, HIDDEN), jnp.float32)
    b2a = b2a.astype(jnp.int32)
    b2revb = b2revb.astype(jnp.int32)

    h = pl.pallas_call(
        _tc_in_body,
        grid=(N_EDGES // BR,),
        in_specs=[pl.BlockSpec((BR, FB), lambda i: (i, 0)),
                  _full((FB, HIDDEN)), _full((1, HIDDEN))],
        out_specs=pl.BlockSpec((BR, HIDDEN), lambda i: (i, 0)),
        out_shape=jax.ShapeDtypeStruct((N_EDGES, HIDDEN), jnp.float32),
    )(f_bonds, W_in_w.astype(jnp.bfloat16), W_in_b.reshape(1, HIDDEN))

    mm_call = pl.pallas_call(
        _tc_mm_body,
        grid=(N_EDGES // BR,),
        in_specs=[pl.BlockSpec((BR, HIDDEN), lambda i: (i, 0)),
                  _full((HIDDEN, HIDDEN))],
        out_specs=pl.BlockSpec((BR, HIDDEN), lambda i: (i, 0)),
        out_shape=jax.ShapeDtypeStruct((N_EDGES, HIDDEN), jnp.float32),
    )

    z_call = pl.pallas_call(
        _tc_z_body,
        grid=(N_ATOMS // BA,),
        in_specs=[pl.BlockSpec((BA, HIDDEN), lambda i: (i, 0)),
                  pl.BlockSpec((BA, HIDDEN), lambda i: (i + N_ATOMS // BA, 0)),
                  _full((HIDDEN, HIDDEN)), _full((1, HIDDEN))],
        out_specs=pl.BlockSpec((BA, HIDDEN), lambda i: (i, 0)),
        out_shape=jax.ShapeDtypeStruct((N_ATOMS, HIDDEN), jnp.float32),
    )

    W_msg_bf = W_msg_w.astype(jnp.bfloat16)
    msg_b = W_msg_b.reshape(1, HIDDEN)
    part = _sc_scatter(h, b2a, b2revb, zeros_a)
    for _ in range(STEPS):
        y = mm_call(h, W_msg_bf)
        z = z_call(part, part, W_msg_w, msg_b)
        h, part = _sc_update_scatter(h, z, y, b2a, b2revb, zeros_a)

    h_atom, node_pred = pl.pallas_call(
        _tc_atom_body,
        grid=(N_ATOMS // BA,),
        in_specs=[pl.BlockSpec((BA, HIDDEN), lambda i: (i, 0)),
                  pl.BlockSpec((BA, HIDDEN), lambda i: (i + N_ATOMS // BA, 0)),
                  pl.BlockSpec((BA, f_atoms.shape[1]), lambda i: (i, 0)),
                  _full((HIDDEN, HIDDEN)), _full((f_atoms.shape[1], HIDDEN)),
                  _full((1, HIDDEN)),
                  _full((HIDDEN, node_w.shape[1])), _full((1, node_w.shape[1]))],
        out_specs=[pl.BlockSpec((BA, HIDDEN), lambda i: (i, 0)),
                   pl.BlockSpec((BA, node_w.shape[1]), lambda i: (i, 0))],
        out_shape=[jax.ShapeDtypeStruct((N_ATOMS, HIDDEN), jnp.float32),
                   jax.ShapeDtypeStruct((N_ATOMS, node_w.shape[1]), jnp.float32)],
    )(part, part, f_atoms, W_atom_w[:HIDDEN], W_atom_w[HIDDEN:],
      W_atom_b.reshape(1, HIDDEN), node_w, node_b.reshape(1, -1))

    edge_pred = pl.pallas_call(
        _tc_edge_body,
        grid=(N_EDGES // BR,),
        in_specs=[pl.BlockSpec((BR, HIDDEN), lambda i: (i, 0)),
                  _full((HIDDEN, edge_w.shape[1])), _full((1, edge_w.shape[1]))],
        out_specs=pl.BlockSpec((BR, edge_w.shape[1]), lambda i: (i, 0)),
        out_shape=jax.ShapeDtypeStruct((N_EDGES, edge_w.shape[1]), jnp.float32),
    )(h, edge_w.astype(jnp.bfloat16), edge_b.reshape(1, -1))

    return (node_pred, edge_pred, h_atom)
